# baseline (speedup 1.0000x reference)
import functools

import jax
import jax.numpy as jnp
from jax import lax
from jax.experimental import pallas as pl
from jax.experimental.pallas import tpu as pltpu
from jax.experimental.pallas import tpu_sc as plsc

NC = 2
NS = 16
NW = NC * NS

BATCH = 16384
HIST = 200
D = 64
HBLK = HIST // 8
BBLK = BATCH // 128
NSB = HBLK * BBLK
SBW = NSB // NW
STEPS = SBW // 2


def _mesh():
    return plsc.VectorSubcoreMesh(
        core_axis_name="c", subcore_axis_name="s",
        num_cores=NC, num_subcores=NS)


@functools.partial(
    pl.kernel,
    out_type=jax.ShapeDtypeStruct((HIST, 8, BBLK, 1024), jnp.float32),
    mesh=_mesh(),
    scratch_types=[
        pltpu.VMEM((2, 8, 128), jnp.int32),
        pltpu.VMEM((2, 128, D), jnp.float32),
        pltpu.VMEM((2, 8, 1024), jnp.float32),
        pltpu.SemaphoreType.DMA((2,)),
        pltpu.SemaphoreType.DMA((2,)),
        pltpu.SemaphoreType.DMA((2,)),
    ],
    compiler_params=pltpu.CompilerParams(use_tc_tiling_on_sc=False,
                                         needs_layout_passes=False),
)
def _gather_kernel(tok4, table, out4, idx_v, rows_v, xp_v,
                   sem_i, sem_g, sem_o):
    wid = lax.axis_index("s") * NC + lax.axis_index("c")
    s0 = wid * SBW

    iota = lax.iota(jnp.int32, 16)
    row_ids = [iota + bb * 16 for bb in range(8)]
    zeros = jnp.zeros((16,), jnp.int32)

    for bi in range(2):
        s = s0 + bi
        pltpu.async_copy(tok4.at[s // BBLK, s % BBLK], idx_v.at[bi],
                         sem_i.at[bi])

    def one_block(j, bi, p, q):
        s = s0 + j
        rb = q

        if q == 0:
            @pl.when(p == 0)
            def _():
                pltpu.async_copy(table.at[idx_v.at[bi, 2 * p]],
                                 rows_v.at[rb], sem_g.at[rb])
        pltpu.make_async_copy(table.at[idx_v.at[bi, 0]], rows_v.at[rb],
                              sem_g.at[rb]).wait()
        if q == 0:
            pltpu.async_copy(table.at[idx_v.at[bi, 2 * p + 1]],
                             rows_v.at[1 - rb], sem_g.at[1 - rb])
        else:
            @pl.when(p < 3)
            def _():
                pltpu.async_copy(table.at[idx_v.at[bi, 2 * p + 2]],
                                 rows_v.at[1 - rb], sem_g.at[1 - rb])

        @pl.when(jnp.logical_or(j > 0, p > 0))
        def _():
            for dblk in range(8):
                pltpu.make_async_copy(
                    xp_v.at[rb, dblk], out4.at[0, dblk, 0],
                    sem_o.at[rb]).wait()

        for dblk in range(8):
            for d8 in range(8):
                col = zeros + (dblk * 8 + d8)
                for bb in range(8):
                    vals = plsc.load_gather(rows_v.at[rb],
                                            [row_ids[bb], col])
                    xp_v[rb, dblk, pl.ds(d8 * 128 + bb * 16, 16)] = vals

        h = (s // BBLK) * 8 + 2 * p + q
        for dblk in range(8):
            pltpu.async_copy(xp_v.at[rb, dblk],
                             out4.at[h, dblk, s % BBLK], sem_o.at[rb])

    def body(j, _):
        s = s0 + j
        bi = lax.rem(j, 2)
        pltpu.make_async_copy(tok4.at[s // BBLK, s % BBLK],
                              idx_v.at[bi], sem_i.at[bi]).wait()

        def pair_body(p, _):
            one_block(j, bi, p, 0)
            one_block(j, bi, p, 1)
            return 0
        lax.fori_loop(0, 4, pair_body, 0)

        @pl.when(j < SBW - 2)
        def _():
            s2 = s + 2
            pltpu.async_copy(tok4.at[s2 // BBLK, s2 % BBLK],
                             idx_v.at[bi], sem_i.at[bi])
        return 0

    lax.fori_loop(0, SBW, body, 0)

    for rb in range(2):
        for dblk in range(8):
            pltpu.make_async_copy(xp_v.at[rb, dblk], out4.at[0, dblk, 0],
                                  sem_o.at[rb]).wait()


def kernel(tokens, embedding):
    tok4 = (tokens.astype(jnp.int32).T
            .reshape(HBLK, 8, BBLK, 128).transpose(0, 2, 1, 3))
    x = _gather_kernel(tok4, embedding)
    return (x.reshape(HIST, 8, BBLK, 8, 128)
            .transpose(2, 4, 0, 1, 3)
            .reshape(BATCH, HIST, D))


# speedup vs baseline: 1.3579x; 1.3579x over previous
import functools

import jax
import jax.numpy as jnp
from jax import lax
from jax.experimental import pallas as pl
from jax.experimental.pallas import tpu as pltpu
from jax.experimental.pallas import tpu_sc as plsc

NC = 2
NS = 16
NW = NC * NS

BATCH = 16384
HIST = 200
D = 64
HBLK = HIST // 8
BBLK = BATCH // 128
NSB = HBLK * BBLK
SBW = NSB // NW
STEPS = SBW // 2


def _mesh():
    return plsc.VectorSubcoreMesh(
        core_axis_name="c", subcore_axis_name="s",
        num_cores=NC, num_subcores=NS)


@functools.partial(
    pl.kernel,
    out_type=jax.ShapeDtypeStruct((HIST, 8, BBLK, 1024), jnp.float32),
    mesh=_mesh(),
    scratch_types=[
        pltpu.VMEM((2, 8, 128), jnp.int32),
        pltpu.VMEM((2, 128, D), jnp.float32),
        pltpu.VMEM((2, 8, 1024), jnp.float32),
        pltpu.SemaphoreType.DMA((2,)),
        pltpu.SemaphoreType.DMA((2,)),
        pltpu.SemaphoreType.DMA((2,)),
    ],
    compiler_params=pltpu.CompilerParams(use_tc_tiling_on_sc=False,
                                         needs_layout_passes=False),
)
def _gather_kernel(tok4, table, out4, idx_v, rows_v, xp_v,
                   sem_i, sem_g, sem_o):
    wid = lax.axis_index("s") * NC + lax.axis_index("c")
    s0 = wid * SBW

    iota = lax.iota(jnp.int32, 16)
    row_ids = [iota + bb * 16 for bb in range(8)]
    zeros = jnp.zeros((16,), jnp.int32)

    for bi in range(2):
        s = s0 + bi
        pltpu.async_copy(tok4.at[s // BBLK, s % BBLK], idx_v.at[bi],
                         sem_i.at[bi])

    def one_block(j, bi, p, q):
        s = s0 + j
        rb = q

        if q == 0:
            @pl.when(p == 0)
            def _():
                pltpu.async_copy(table.at[idx_v.at[bi, 2 * p]],
                                 rows_v.at[rb], sem_g.at[rb])
        pltpu.make_async_copy(table.at[idx_v.at[bi, 0]], rows_v.at[rb],
                              sem_g.at[rb]).wait()
        if q == 0:
            pltpu.async_copy(table.at[idx_v.at[bi, 2 * p + 1]],
                             rows_v.at[1 - rb], sem_g.at[1 - rb])
        else:
            @pl.when(p < 3)
            def _():
                pltpu.async_copy(table.at[idx_v.at[bi, 2 * p + 2]],
                                 rows_v.at[1 - rb], sem_g.at[1 - rb])

        @pl.when(jnp.logical_or(j > 0, p > 0))
        def _():
            for dblk in range(8):
                pltpu.make_async_copy(
                    xp_v.at[rb, dblk], out4.at[0, dblk, 0],
                    sem_o.at[rb]).wait()

        for dblk in range(8):
            for d8 in range(8):
                col = zeros + (dblk * 8 + d8)
                vals = [plsc.load_gather(rows_v.at[rb], [row_ids[bb], col])
                        for bb in range(8)]
                for bb in range(8):
                    xp_v[rb, dblk,
                         pl.ds(d8 * 128 + bb * 16, 16)] = vals[bb]

        h = (s // BBLK) * 8 + 2 * p + q
        for dblk in range(8):
            pltpu.async_copy(xp_v.at[rb, dblk],
                             out4.at[h, dblk, s % BBLK], sem_o.at[rb])

    def body(j, _):
        s = s0 + j
        bi = lax.rem(j, 2)
        pltpu.make_async_copy(tok4.at[s // BBLK, s % BBLK],
                              idx_v.at[bi], sem_i.at[bi]).wait()

        def pair_body(p, _):
            one_block(j, bi, p, 0)
            one_block(j, bi, p, 1)
            return 0
        lax.fori_loop(0, 4, pair_body, 0)

        @pl.when(j < SBW - 2)
        def _():
            s2 = s + 2
            pltpu.async_copy(tok4.at[s2 // BBLK, s2 % BBLK],
                             idx_v.at[bi], sem_i.at[bi])
        return 0

    lax.fori_loop(0, SBW, body, 0)

    for rb in range(2):
        for dblk in range(8):
            pltpu.make_async_copy(xp_v.at[rb, dblk], out4.at[0, dblk, 0],
                                  sem_o.at[rb]).wait()


def kernel(tokens, embedding):
    tok4 = (tokens.astype(jnp.int32).T
            .reshape(HBLK, 8, BBLK, 128).transpose(0, 2, 1, 3))
    x = _gather_kernel(tok4, embedding)
    return (x.reshape(HIST, 8, BBLK, 8, 128)
            .transpose(2, 4, 0, 1, 3)
            .reshape(BATCH, HIST, D))
